# baseline (speedup 1.0000x reference)
import functools

import jax
import jax.numpy as jnp
from jax import lax
from jax.experimental import pallas as pl
from jax.experimental.pallas import tpu as pltpu
from jax.experimental.pallas import tpu_sc as plsc

MARGIN = 0.1


@functools.lru_cache(maxsize=None)
def _sc_gather(T, B, N):
    info = plsc.get_sparse_core_info()
    NC, NS, L = info.num_cores, info.num_subcores, info.num_lanes
    NW = NC * NS
    SR = N // 128
    assert NW >= B and N % 128 == 0 and 128 % L == 0
    assert B % 8 == 0 and B <= L
    mesh = plsc.VectorSubcoreMesh(core_axis_name="c", subcore_axis_name="s")

    @functools.partial(
        pl.kernel,
        mesh=mesh,
        compiler_params=pltpu.CompilerParams(needs_layout_passes=False),
        out_type=jax.ShapeDtypeStruct((B, SR, 128), jnp.float32),
        scratch_types=[
            pltpu.VMEM((T,), jnp.float32),
            pltpu.VMEM((L,), jnp.int32),
            pltpu.VMEM((L,), jnp.int32),
            pltpu.VMEM((N,), jnp.int32),
            pltpu.VMEM((SR, 128), jnp.float32),
        ],
    )
    def sc_gather(scores_hbm, nbest_hbm, rank_hbm, out_hbm,
                  scores_v, nb_v, tmp_v, idx_v, g_v):
        wid = lax.axis_index("s") * NC + lax.axis_index("c")

        @pl.when(wid < B)
        def _():
            b = wid
            pltpu.sync_copy(scores_hbm, scores_v)
            pltpu.sync_copy(nbest_hbm, nb_v.at[pl.ds(0, B)])
            for tc in range(N // 128):
                pltpu.sync_copy(
                    rank_hbm.at[b // 8, tc, b % 8, :],
                    idx_v.at[pl.ds(tc * 128, 128)],
                )
            lane = lax.iota(jnp.int32, L)
            nb = jnp.where((lane < b) & (lane < B), nb_v[...], 0)
            for sh in (8, 4, 2, 1):
                tmp_v[...] = nb
                nb = nb + plsc.load_gather(tmp_v, [lane ^ sh])
            off_b = nb
            for k in range(N // L):
                v = idx_v[pl.ds(k * L, L)] + off_b
                g_v[(k * L) // 128, pl.ds((k * L) % 128, L)] = (
                    plsc.load_gather(scores_v, [v])
                )
            pltpu.sync_copy(g_v, out_hbm.at[b])

    return sc_gather


@functools.lru_cache(maxsize=None)
def _tc_loss(B, N, interpret=False):
    SR = N // 128
    assert N % 128 == 0

    def body(g_ref, out_ref):
        b = pl.program_id(0)
        gmat = g_ref[0]
        gt = jnp.transpose(gmat)
        acc = jnp.zeros((128, 128), jnp.float32)
        for jt in range(SR):
            rowm = gmat[jt:jt + 1, :] + MARGIN
            for it in range(jt + 1):
                col = gt[:, it:it + 1]
                d = jnp.maximum(rowm - col, 0.0)
                if it == jt:
                    li = lax.broadcasted_iota(jnp.int32, (128, 128), 0)
                    lj = lax.broadcasted_iota(jnp.int32, (128, 128), 1)
                    d = jnp.where(lj > li, d, 0.0)
                acc = acc + d
        total = jnp.sum(acc)

        @pl.when(b == 0)
        def _init():
            out_ref[...] = jnp.zeros((1, 1), jnp.float32)

        out_ref[...] += jnp.reshape(total, (1, 1))

    return pl.pallas_call(
        body,
        grid=(B,),
        in_specs=[pl.BlockSpec((1, SR, 128), lambda b: (b, 0, 0))],
        out_specs=pl.BlockSpec((1, 1), lambda b: (0, 0)),
        out_shape=jax.ShapeDtypeStruct((1, 1), jnp.float32),
        interpret=interpret,
    )


def kernel(scores, nBestIndex, werRank):
    B, N = werRank.shape
    T = scores.shape[0]
    xt = jnp.transpose(
        werRank.astype(jnp.int32).reshape(B // 8, 8, N // 128, 128),
        (0, 2, 1, 3),
    )
    g = _sc_gather(T, B, N)(scores, nBestIndex.astype(jnp.int32), xt)
    loss = _tc_loss(B, N)(g)
    return loss[0, 0]


# speedup vs baseline: 1.0968x; 1.0968x over previous
import functools

import jax
import jax.numpy as jnp
from jax import lax
from jax.experimental import pallas as pl
from jax.experimental.pallas import tpu as pltpu
from jax.experimental.pallas import tpu_sc as plsc

MARGIN = 0.1


@functools.lru_cache(maxsize=None)
def _sc_gather(T, B, N):
    info = plsc.get_sparse_core_info()
    NC, NS, L = info.num_cores, info.num_subcores, info.num_lanes
    NW = NC * NS
    SR = N // 128
    assert NW >= B and N % 128 == 0 and 128 % L == 0
    assert B % 8 == 0 and B <= L
    mesh = plsc.VectorSubcoreMesh(core_axis_name="c", subcore_axis_name="s")

    @functools.partial(
        pl.kernel,
        mesh=mesh,
        compiler_params=pltpu.CompilerParams(needs_layout_passes=False),
        out_type=jax.ShapeDtypeStruct((B, SR, 128), jnp.float32),
        scratch_types=[
            pltpu.VMEM((T,), jnp.float32),
            pltpu.VMEM((L,), jnp.int32),
            pltpu.VMEM((L,), jnp.int32),
            pltpu.VMEM((N,), jnp.int32),
            pltpu.VMEM((SR, 128), jnp.float32),
        ],
    )
    def sc_gather(scores_hbm, nbest_hbm, rank_hbm, out_hbm,
                  scores_v, nb_v, tmp_v, idx_v, g_v):
        wid = lax.axis_index("s") * NC + lax.axis_index("c")

        @pl.when(wid < B)
        def _():
            b = wid
            pltpu.sync_copy(scores_hbm, scores_v)
            pltpu.sync_copy(nbest_hbm, nb_v.at[pl.ds(0, B)])
            pltpu.sync_copy(rank_hbm.at[pl.ds(b * N, N)], idx_v)
            lane = lax.iota(jnp.int32, L)
            nb = jnp.where((lane < b) & (lane < B), nb_v[...], 0)
            for sh in (8, 4, 2, 1):
                tmp_v[...] = nb
                nb = nb + plsc.load_gather(tmp_v, [lane ^ sh])
            off_b = nb
            for k in range(N // L):
                v = idx_v[pl.ds(k * L, L)] + off_b
                g_v[(k * L) // 128, pl.ds((k * L) % 128, L)] = (
                    plsc.load_gather(scores_v, [v])
                )
            pltpu.sync_copy(g_v, out_hbm.at[b])

    return sc_gather


@functools.lru_cache(maxsize=None)
def _tc_loss(B, N, interpret=False):
    SR = N // 128
    assert N % 128 == 0

    def body(g_ref, out_ref):
        b = pl.program_id(0)
        gmat = g_ref[0]
        gt = jnp.transpose(gmat)
        acc = jnp.zeros((128, 128), jnp.float32)
        for jt in range(SR):
            rowm = gmat[jt:jt + 1, :] + MARGIN
            for it in range(jt + 1):
                col = gt[:, it:it + 1]
                d = jnp.maximum(rowm - col, 0.0)
                if it == jt:
                    li = lax.broadcasted_iota(jnp.int32, (128, 128), 0)
                    lj = lax.broadcasted_iota(jnp.int32, (128, 128), 1)
                    d = jnp.where(lj > li, d, 0.0)
                acc = acc + d
        total = jnp.sum(acc)

        @pl.when(b == 0)
        def _init():
            out_ref[...] = jnp.zeros((1, 1), jnp.float32)

        out_ref[...] += jnp.reshape(total, (1, 1))

    return pl.pallas_call(
        body,
        grid=(B,),
        in_specs=[pl.BlockSpec((1, SR, 128), lambda b: (b, 0, 0))],
        out_specs=pl.BlockSpec((1, 1), lambda b: (0, 0)),
        out_shape=jax.ShapeDtypeStruct((1, 1), jnp.float32),
        interpret=interpret,
    )


def kernel(scores, nBestIndex, werRank):
    B, N = werRank.shape
    T = scores.shape[0]
    g = _sc_gather(T, B, N)(
        scores, nBestIndex.astype(jnp.int32),
        werRank.astype(jnp.int32).reshape(B * N),
    )
    loss = _tc_loss(B, N)(g)
    return loss[0, 0]


# speedup vs baseline: 1.1005x; 1.0034x over previous
import functools

import jax
import jax.numpy as jnp
from jax import lax
from jax.experimental import pallas as pl
from jax.experimental.pallas import tpu as pltpu
from jax.experimental.pallas import tpu_sc as plsc

MARGIN = 0.1


@functools.lru_cache(maxsize=None)
def _sc_gather(T, B, N):
    info = plsc.get_sparse_core_info()
    NC, NS, L = info.num_cores, info.num_subcores, info.num_lanes
    NW = NC * NS
    SR = N // 128
    assert NW >= B and N % 128 == 0 and 128 % L == 0
    assert B % 8 == 0 and B <= L
    mesh = plsc.VectorSubcoreMesh(core_axis_name="c", subcore_axis_name="s")

    @functools.partial(
        pl.kernel,
        mesh=mesh,
        compiler_params=pltpu.CompilerParams(needs_layout_passes=False),
        out_type=jax.ShapeDtypeStruct((B, SR, 128), jnp.float32),
        scratch_types=[
            pltpu.VMEM((T,), jnp.float32),
            pltpu.VMEM((L,), jnp.int32),
            pltpu.VMEM((L,), jnp.int32),
            pltpu.VMEM((N // 128, 128), jnp.int32),
            pltpu.VMEM((SR, 128), jnp.float32),
        ],
    )
    def sc_gather(scores_hbm, nbest_hbm, rank_hbm, out_hbm,
                  scores_v, nb_v, tmp_v, idx_v, g_v):
        wid = lax.axis_index("s") * NC + lax.axis_index("c")

        @pl.when(wid < B)
        def _():
            b = wid
            pltpu.sync_copy(scores_hbm, scores_v)
            pltpu.sync_copy(nbest_hbm, nb_v.at[pl.ds(0, B)])
            pltpu.sync_copy(rank_hbm.at[b // 8, :, b % 8, :], idx_v)
            lane = lax.iota(jnp.int32, L)
            nb = jnp.where((lane < b) & (lane < B), nb_v[...], 0)
            for sh in (8, 4, 2, 1):
                tmp_v[...] = nb
                nb = nb + plsc.load_gather(tmp_v, [lane ^ sh])
            off_b = nb
            for k in range(N // L):
                v = idx_v[(k * L) // 128, pl.ds((k * L) % 128, L)] + off_b
                g_v[(k * L) // 128, pl.ds((k * L) % 128, L)] = (
                    plsc.load_gather(scores_v, [v])
                )
            pltpu.sync_copy(g_v, out_hbm.at[b])

    return sc_gather


@functools.lru_cache(maxsize=None)
def _tc_loss(B, N, interpret=False):
    SR = N // 128
    assert N % 128 == 0

    def body(g_ref, out_ref):
        b = pl.program_id(0)
        gmat = g_ref[0]
        gt = jnp.transpose(gmat)
        acc = jnp.zeros((128, 128), jnp.float32)
        for jt in range(SR):
            rowm = gmat[jt:jt + 1, :] + MARGIN
            for it in range(jt + 1):
                col = gt[:, it:it + 1]
                d = jnp.maximum(rowm - col, 0.0)
                if it == jt:
                    li = lax.broadcasted_iota(jnp.int32, (128, 128), 0)
                    lj = lax.broadcasted_iota(jnp.int32, (128, 128), 1)
                    d = jnp.where(lj > li, d, 0.0)
                acc = acc + d
        total = jnp.sum(acc)

        @pl.when(b == 0)
        def _init():
            out_ref[...] = jnp.zeros((1, 1), jnp.float32)

        out_ref[...] += jnp.reshape(total, (1, 1))

    return pl.pallas_call(
        body,
        grid=(B,),
        in_specs=[pl.BlockSpec((1, SR, 128), lambda b: (b, 0, 0))],
        out_specs=pl.BlockSpec((1, 1), lambda b: (0, 0)),
        out_shape=jax.ShapeDtypeStruct((1, 1), jnp.float32),
        interpret=interpret,
    )


def kernel(scores, nBestIndex, werRank):
    B, N = werRank.shape
    T = scores.shape[0]
    xt = jnp.transpose(
        werRank.astype(jnp.int32).reshape(B // 8, 8, N // 128, 128),
        (0, 2, 1, 3),
    )
    g = _sc_gather(T, B, N)(scores, nBestIndex.astype(jnp.int32), xt)
    loss = _tc_loss(B, N)(g)
    return loss[0, 0]


# speedup vs baseline: 1.1281x; 1.0251x over previous
import functools

import jax
import jax.numpy as jnp
from jax import lax
from jax.experimental import pallas as pl
from jax.experimental.pallas import tpu as pltpu
from jax.experimental.pallas import tpu_sc as plsc

MARGIN = 0.1


@functools.lru_cache(maxsize=None)
def _sc_gather(T, B, N):
    info = plsc.get_sparse_core_info()
    NC, NS, L = info.num_cores, info.num_subcores, info.num_lanes
    NW = NC * NS
    SR = N // 128
    assert NW >= B and N % 128 == 0 and 128 % L == 0
    assert B % 8 == 0 and B <= L
    NC = 1 if NS >= B else NC
    NW = NC * NS
    mesh = plsc.VectorSubcoreMesh(
        core_axis_name="c", subcore_axis_name="s", num_cores=NC
    )

    @functools.partial(
        pl.kernel,
        mesh=mesh,
        compiler_params=pltpu.CompilerParams(needs_layout_passes=False),
        out_type=jax.ShapeDtypeStruct((B, SR, 128), jnp.float32),
        scratch_types=[
            pltpu.VMEM((T,), jnp.float32),
            pltpu.VMEM((L,), jnp.int32),
            pltpu.VMEM((L,), jnp.int32),
            pltpu.VMEM((N // 128, 128), jnp.int32),
            pltpu.VMEM((SR, 128), jnp.float32),
        ],
    )
    def sc_gather(scores_hbm, nbest_hbm, rank_hbm, out_hbm,
                  scores_v, nb_v, tmp_v, idx_v, g_v):
        wid = lax.axis_index("s") * NC + lax.axis_index("c")

        @pl.when(wid < B)
        def _():
            b = wid
            pltpu.sync_copy(scores_hbm, scores_v)
            pltpu.sync_copy(nbest_hbm, nb_v.at[pl.ds(0, B)])
            pltpu.sync_copy(rank_hbm.at[b // 8, :, b % 8, :], idx_v)
            lane = lax.iota(jnp.int32, L)
            nb = jnp.where((lane < b) & (lane < B), nb_v[...], 0)
            for sh in (8, 4, 2, 1):
                tmp_v[...] = nb
                nb = nb + plsc.load_gather(tmp_v, [lane ^ sh])
            off_b = nb
            for k in range(N // L):
                v = idx_v[(k * L) // 128, pl.ds((k * L) % 128, L)] + off_b
                g_v[(k * L) // 128, pl.ds((k * L) % 128, L)] = (
                    plsc.load_gather(scores_v, [v])
                )
            pltpu.sync_copy(g_v, out_hbm.at[b])

    return sc_gather


@functools.lru_cache(maxsize=None)
def _tc_loss(B, N, interpret=False):
    SR = N // 128
    assert N % 128 == 0

    def body(g_ref, out_ref):
        b = pl.program_id(0)
        gmat = g_ref[0]
        gt = jnp.transpose(gmat)
        acc = jnp.zeros((128, 128), jnp.float32)
        for jt in range(SR):
            rowm = gmat[jt:jt + 1, :] + MARGIN
            for it in range(jt + 1):
                col = gt[:, it:it + 1]
                d = jnp.maximum(rowm - col, 0.0)
                if it == jt:
                    li = lax.broadcasted_iota(jnp.int32, (128, 128), 0)
                    lj = lax.broadcasted_iota(jnp.int32, (128, 128), 1)
                    d = jnp.where(lj > li, d, 0.0)
                acc = acc + d
        total = jnp.sum(acc)

        @pl.when(b == 0)
        def _init():
            out_ref[...] = jnp.zeros((1, 1), jnp.float32)

        out_ref[...] += jnp.reshape(total, (1, 1))

    return pl.pallas_call(
        body,
        grid=(B,),
        in_specs=[pl.BlockSpec((1, SR, 128), lambda b: (b, 0, 0))],
        out_specs=pl.BlockSpec((1, 1), lambda b: (0, 0)),
        out_shape=jax.ShapeDtypeStruct((1, 1), jnp.float32),
        interpret=interpret,
    )


def kernel(scores, nBestIndex, werRank):
    B, N = werRank.shape
    T = scores.shape[0]
    xt = jnp.transpose(
        werRank.astype(jnp.int32).reshape(B // 8, 8, N // 128, 128),
        (0, 2, 1, 3),
    )
    g = _sc_gather(T, B, N)(scores, nBestIndex.astype(jnp.int32), xt)
    loss = _tc_loss(B, N)(g)
    return loss[0, 0]


# speedup vs baseline: 1.2356x; 1.0953x over previous
import functools

import jax
import jax.numpy as jnp
from jax import lax
from jax.experimental import pallas as pl
from jax.experimental.pallas import tpu as pltpu
from jax.experimental.pallas import tpu_sc as plsc

MARGIN = 0.1


@functools.lru_cache(maxsize=None)
def _sc_gather(T, B, N):
    info = plsc.get_sparse_core_info()
    NC, NS, L = info.num_cores, info.num_subcores, info.num_lanes
    NW = NC * NS
    SR = N // 128
    assert NW >= B and N % 128 == 0 and 128 % L == 0
    assert B % 8 == 0 and B <= L
    NC = 1 if NS >= B else NC
    NW = NC * NS
    mesh = plsc.VectorSubcoreMesh(
        core_axis_name="c", subcore_axis_name="s", num_cores=NC
    )

    @functools.partial(
        pl.kernel,
        mesh=mesh,
        compiler_params=pltpu.CompilerParams(needs_layout_passes=False),
        out_type=jax.ShapeDtypeStruct((B, SR, 128), jnp.float32),
        scratch_types=[
            pltpu.VMEM((T,), jnp.float32),
            pltpu.VMEM((L,), jnp.int32),
            pltpu.VMEM((L,), jnp.int32),
            pltpu.VMEM((N // 128, 128), jnp.int32),
            pltpu.VMEM((SR, 128), jnp.float32),
        ],
    )
    def sc_gather(scores_hbm, nbest_hbm, rank_hbm, out_hbm,
                  scores_v, nb_v, tmp_v, idx_v, g_v):
        wid = lax.axis_index("s") * NC + lax.axis_index("c")

        @pl.when(wid < B)
        def _():
            b = wid
            pltpu.sync_copy(scores_hbm, scores_v)
            pltpu.sync_copy(nbest_hbm, nb_v.at[pl.ds(0, B)])
            pltpu.sync_copy(rank_hbm.at[b // 8, :, b % 8, :], idx_v)
            lane = lax.iota(jnp.int32, L)
            nb = jnp.where((lane < b) & (lane < B), nb_v[...], 0)
            for sh in (8, 4, 2, 1):
                tmp_v[...] = nb
                nb = nb + plsc.load_gather(tmp_v, [lane ^ sh])
            off_b = nb
            for k in range(N // L):
                v = idx_v[(k * L) // 128, pl.ds((k * L) % 128, L)] + off_b
                g_v[(k * L) // 128, pl.ds((k * L) % 128, L)] = (
                    plsc.load_gather(scores_v, [v])
                )
            pltpu.sync_copy(g_v, out_hbm.at[b])

    return sc_gather


@functools.lru_cache(maxsize=None)
def _tc_loss(B, N, interpret=False):
    SR = N // 128
    assert N % 128 == 0

    UB = 2 if B % 2 == 0 else 1

    def body(g_ref, out_ref):
        b = pl.program_id(0)
        total = 0.0
        for u in range(UB):
            gmat = g_ref[u]
            gt = jnp.transpose(gmat)
            acc = jnp.zeros((128, 128), jnp.float32)
            for jt in range(SR):
                rowm = gmat[jt:jt + 1, :] + MARGIN
                for it in range(jt + 1):
                    col = gt[:, it:it + 1]
                    d = jnp.maximum(rowm - col, 0.0)
                    if it == jt:
                        li = lax.broadcasted_iota(jnp.int32, (128, 128), 0)
                        lj = lax.broadcasted_iota(jnp.int32, (128, 128), 1)
                        d = jnp.where(lj > li, d, 0.0)
                    acc = acc + d
            total = total + jnp.sum(acc)

        @pl.when(b == 0)
        def _init():
            out_ref[...] = jnp.zeros((1, 1), jnp.float32)

        out_ref[...] += jnp.reshape(total, (1, 1))

    return pl.pallas_call(
        body,
        grid=(B // UB,),
        in_specs=[pl.BlockSpec((UB, SR, 128), lambda b: (b, 0, 0))],
        out_specs=pl.BlockSpec((1, 1), lambda b: (0, 0)),
        out_shape=jax.ShapeDtypeStruct((1, 1), jnp.float32),
        interpret=interpret,
    )


def kernel(scores, nBestIndex, werRank):
    B, N = werRank.shape
    T = scores.shape[0]
    xt = jnp.transpose(
        werRank.astype(jnp.int32).reshape(B // 8, 8, N // 128, 128),
        (0, 2, 1, 3),
    )
    g = _sc_gather(T, B, N)(scores, nBestIndex.astype(jnp.int32), xt)
    loss = _tc_loss(B, N)(g)
    return loss[0, 0]


# speedup vs baseline: 1.2579x; 1.0180x over previous
import functools

import jax
import jax.numpy as jnp
from jax import lax
from jax.experimental import pallas as pl
from jax.experimental.pallas import tpu as pltpu
from jax.experimental.pallas import tpu_sc as plsc

MARGIN = 0.1


@functools.lru_cache(maxsize=None)
def _sc_gather(T, B, N):
    info = plsc.get_sparse_core_info()
    NC, NS, L = info.num_cores, info.num_subcores, info.num_lanes
    NW = NC * NS
    SR = N // 128
    assert NW >= B and N % 128 == 0 and 128 % L == 0
    assert B % 8 == 0 and B <= L
    NC = 1 if NS >= B else NC
    NW = NC * NS
    mesh = plsc.VectorSubcoreMesh(
        core_axis_name="c", subcore_axis_name="s", num_cores=NC
    )

    @functools.partial(
        pl.kernel,
        mesh=mesh,
        compiler_params=pltpu.CompilerParams(needs_layout_passes=False),
        out_type=jax.ShapeDtypeStruct((B, SR, 128), jnp.float32),
        scratch_types=[
            pltpu.VMEM((T,), jnp.float32),
            pltpu.VMEM((L,), jnp.int32),
            pltpu.VMEM((L,), jnp.int32),
            pltpu.VMEM((N // 128, 128), jnp.int32),
            pltpu.VMEM((SR, 128), jnp.float32),
        ],
    )
    def sc_gather(scores_hbm, nbest_hbm, rank_hbm, out_hbm,
                  scores_v, nb_v, tmp_v, idx_v, g_v):
        wid = lax.axis_index("s") * NC + lax.axis_index("c")

        @pl.when(wid < B)
        def _():
            b = wid
            pltpu.sync_copy(scores_hbm, scores_v)
            pltpu.sync_copy(nbest_hbm, nb_v.at[pl.ds(0, B)])
            pltpu.sync_copy(rank_hbm.at[b // 8, :, b % 8, :], idx_v)
            lane = lax.iota(jnp.int32, L)
            nb = jnp.where((lane < b) & (lane < B), nb_v[...], 0)
            for sh in (8, 4, 2, 1):
                tmp_v[...] = nb
                nb = nb + plsc.load_gather(tmp_v, [lane ^ sh])
            off_b = nb
            for k in range(N // L):
                v = idx_v[(k * L) // 128, pl.ds((k * L) % 128, L)] + off_b
                g_v[(k * L) // 128, pl.ds((k * L) % 128, L)] = (
                    plsc.load_gather(scores_v, [v])
                )
            pltpu.sync_copy(g_v, out_hbm.at[b])

    return sc_gather


@functools.lru_cache(maxsize=None)
def _tc_loss(B, N, interpret=False):
    SR = N // 128
    assert N % 128 == 0

    UB = 4 if B % 4 == 0 else (2 if B % 2 == 0 else 1)

    def body(g_ref, out_ref):
        b = pl.program_id(0)
        total = 0.0
        for u in range(UB):
            gmat = g_ref[u]
            gt = jnp.transpose(gmat)
            acc = jnp.zeros((128, 128), jnp.float32)
            for jt in range(SR):
                rowm = gmat[jt:jt + 1, :] + MARGIN
                for it in range(jt + 1):
                    col = gt[:, it:it + 1]
                    d = jnp.maximum(rowm - col, 0.0)
                    if it == jt:
                        li = lax.broadcasted_iota(jnp.int32, (128, 128), 0)
                        lj = lax.broadcasted_iota(jnp.int32, (128, 128), 1)
                        d = jnp.where(lj > li, d, 0.0)
                    acc = acc + d
            total = total + jnp.sum(acc)

        @pl.when(b == 0)
        def _init():
            out_ref[...] = jnp.zeros((1, 1), jnp.float32)

        out_ref[...] += jnp.reshape(total, (1, 1))

    return pl.pallas_call(
        body,
        grid=(B // UB,),
        in_specs=[pl.BlockSpec((UB, SR, 128), lambda b: (b, 0, 0))],
        out_specs=pl.BlockSpec((1, 1), lambda b: (0, 0)),
        out_shape=jax.ShapeDtypeStruct((1, 1), jnp.float32),
        interpret=interpret,
    )


def kernel(scores, nBestIndex, werRank):
    B, N = werRank.shape
    T = scores.shape[0]
    xt = jnp.transpose(
        werRank.astype(jnp.int32).reshape(B // 8, 8, N // 128, 128),
        (0, 2, 1, 3),
    )
    g = _sc_gather(T, B, N)(scores, nBestIndex.astype(jnp.int32), xt)
    loss = _tc_loss(B, N)(g)
    return loss[0, 0]
